# baseline (speedup 1.0000x reference)
import functools

import jax
import jax.numpy as jnp
from jax import lax
from jax.experimental import pallas as pl
from jax.experimental.pallas import tpu as pltpu
from jax.experimental.pallas import tpu_sc as plsc

B = 16384
V = 1000000
R = 1000
D = 64

NC = 2
NS = 16
L = 16
NW = NC * NS
BPW = B // NW
CHUNK = 128
NCHUNK = BPW // CHUNK
NG = BPW // L


def _rsqrt(x):
    i = plsc.bitcast(x, jnp.int32)
    y = plsc.bitcast(jnp.int32(0x5F3759DF) - (i >> 1), jnp.float32)
    xh = x * 0.5
    for _ in range(3):
        y = y * (1.5 - (xh * y) * y)
    return y


def _trans_e_body(head_hbm, label_hbm, tail_hbm, ent_hbm, rel_hbm, out_hbm,
                  idx_h, idx_l, idx_t, hrows, trows, rrows, outv, sem):
    wid = lax.axis_index("s") * NC + lax.axis_index("c")

    pltpu.sync_copy(head_hbm.at[wid], idx_h)
    pltpu.sync_copy(label_hbm.at[wid], idx_l)
    pltpu.sync_copy(tail_hbm.at[wid], idx_t)

    descs = []
    for c in range(NCHUNK):
        rows = pl.ds(c * CHUNK, CHUNK)
        descs.append(pltpu.async_copy(ent_hbm.at[idx_h.at[c]], hrows.at[rows], sem))
        descs.append(pltpu.async_copy(ent_hbm.at[idx_t.at[c]], trows.at[rows], sem))
        descs.append(pltpu.async_copy(rel_hbm.at[idx_l.at[c]], rrows.at[rows], sem))
    for d in descs:
        d.wait()

    def group(g, carry):
        row = g * L + lax.iota(jnp.int32, L)
        zero = jnp.zeros((L,), jnp.float32)
        hh = zero; tt = zero; rr = zero
        hr = zero; ht = zero; rt = zero
        for j in range(D):
            col = jnp.full((L,), j, jnp.int32)
            h = plsc.load_gather(hrows, [row, col])
            t = plsc.load_gather(trows, [row, col])
            r = plsc.load_gather(rrows, [row, col])
            hh = hh + h * h
            tt = tt + t * t
            rr = rr + r * r
            hr = hr + h * r
            ht = ht + h * t
            rt = rt + t * r
        a = 1.0 / jnp.maximum(hh * _rsqrt(hh), 1e-12)
        b = 1.0 / jnp.maximum(tt * _rsqrt(tt), 1e-12)
        d2 = ((a * hh) * a + rr + (b * tt) * b
              + 2.0 * (a * hr) - 2.0 * ((a * ht) * b) - 2.0 * (b * rt))
        d2 = jnp.maximum(d2, 0.0)
        plsc.store_scatter(outv, [row], d2 * _rsqrt(d2))
        return carry

    lax.fori_loop(0, NG, group, 0)

    pltpu.sync_copy(outv, out_hbm.at[pl.ds(wid * BPW, BPW)])


@jax.jit
def kernel(head_ind, label, tail_ind, ent_embs, rel_embs):
    mesh = plsc.VectorSubcoreMesh(core_axis_name="c", subcore_axis_name="s")
    run = pl.kernel(
        _trans_e_body,
        mesh=mesh,
        compiler_params=pltpu.CompilerParams(needs_layout_passes=False,
                                              use_tc_tiling_on_sc=False),
        out_type=jax.ShapeDtypeStruct((B,), jnp.float32),
        scratch_types=[
            pltpu.VMEM((NCHUNK, CHUNK), jnp.int32),
            pltpu.VMEM((NCHUNK, CHUNK), jnp.int32),
            pltpu.VMEM((NCHUNK, CHUNK), jnp.int32),
            pltpu.VMEM((BPW, D), jnp.float32),
            pltpu.VMEM((BPW, D), jnp.float32),
            pltpu.VMEM((BPW, D), jnp.float32),
            pltpu.VMEM((BPW,), jnp.float32),
            pltpu.SemaphoreType.DMA,
        ],
    )
    h3 = head_ind.astype(jnp.int32).reshape(NW, NCHUNK, CHUNK)
    l3 = label.astype(jnp.int32).reshape(NW, NCHUNK, CHUNK)
    t3 = tail_ind.astype(jnp.int32).reshape(NW, NCHUNK, CHUNK)
    return run(h3, l3, t3, ent_embs, rel_embs)
